# baseline (speedup 1.0000x reference)
import functools

import jax
import jax.numpy as jnp
from jax import lax
from jax.experimental import pallas as pl
from jax.experimental.pallas import tpu as pltpu
from jax.experimental.pallas import tpu_sc as plsc

N = 5000
TH = 0.5
L = 16
NS = 16
B = 128
N_PAD = 5120
NB = N_PAD // B
NCB = B // L
C = N_PAD // L
PAD = -1e30


def _nms_body(x1h, y1h, x2h, y2h, oh, outh,
              x1v, y1v, x2v, y2v, arv, odv, stv, shv):
    sid = lax.axis_index("s")

    pltpu.sync_copy(x1h, x1v)
    pltpu.sync_copy(y1h, y1v)
    pltpu.sync_copy(x2h, x2v)
    pltpu.sync_copy(y2h, y2v)
    pltpu.sync_copy(oh, odv)

    def init_c(c, _):
        o = c * L
        w = jnp.maximum(x2v[pl.ds(o, L)] - x1v[pl.ds(o, L)], 0.0)
        h = jnp.maximum(y2v[pl.ds(o, L)] - y1v[pl.ds(o, L)], 0.0)
        arv[pl.ds(o, L)] = w * h
        return 0

    lax.fori_loop(0, N_PAD // L, init_c, 0)
    lanes = lax.iota(jnp.int32, L)

    def block_body(k, _):
        base = k * B
        j = jnp.remainder(sid - k * NCB, NS)

        @pl.when(j < NCB)
        def _():
            pltpu.sync_copy(arv.at[pl.ds(base + j * L, L)],
                            shv.at[pl.ds(j * L, L)])

        plsc.subcore_barrier()
        pltpu.sync_copy(shv, arv.at[pl.ds(base, B)])
        plsc.subcore_barrier()

        start = (k + 1) * NCB
        c0 = start + jnp.remainder(sid - start, NS)

        def chunk_body(cc, _):
            ci = k * NCB + cc
            oi = ci * L
            x1c = x1v[pl.ds(oi, L)]
            y1c = y1v[pl.ds(oi, L)]
            x2c = x2v[pl.ds(oi, L)]
            y2c = y2v[pl.ds(oi, L)]
            arc = arv[pl.ds(oi, L)]

            def lane_body(li, _):
                gi = oi + li
                a_i = arv[pl.ds(gi, L)][0]

                @pl.when(a_i > 0.0)
                def _():
                    liv = jnp.full((L,), li, jnp.int32)

                    def tk(vec):
                        return vec.at[liv].get(mode="promise_in_bounds")

                    x1i = tk(x1c)
                    y1i = tk(y1c)
                    x2i = tk(x2c)
                    y2i = tk(y2c)
                    ari = tk(arc)

                    def sup_chunk(c, extra=None):
                        o = c * L
                        ix1 = jnp.maximum(x1v[pl.ds(o, L)], x1i)
                        iy1 = jnp.maximum(y1v[pl.ds(o, L)], y1i)
                        ix2 = jnp.minimum(x2v[pl.ds(o, L)], x2i)
                        iy2 = jnp.minimum(y2v[pl.ds(o, L)], y2i)
                        inter = (jnp.maximum(ix2 - ix1, 0.0)
                                 * jnp.maximum(iy2 - iy1, 0.0))
                        arj = arv[pl.ds(o, L)]
                        aj = jnp.abs(arj)
                        union = aj + ari - inter
                        sup = inter > union * TH
                        if extra is not None:
                            sup = jnp.logical_and(sup, extra)
                        arv[pl.ds(o, L)] = jnp.where(sup, -aj, arj)

                    sup_chunk(ci, lanes > li)

                    @plsc.parallel_loop(ci + 1, (k + 1) * NCB, unroll=4)
                    def _rest(c):
                        sup_chunk(c)

                    @plsc.parallel_loop(c0, C, step=NS, unroll=4)
                    def _tail(c):
                        sup_chunk(c)

                return 0

            lax.fori_loop(0, L, lane_body, 0)
            return 0

        lax.fori_loop(0, NCB, chunk_body, 0)
        return 0

    lax.fori_loop(0, NB, block_body, 0)

    def out_body(m, _):
        o = (m * NS + sid) * L
        stv[...] = jnp.where(arv[pl.ds(o, L)] > 0.0, odv[pl.ds(o, L)],
                             jnp.full((L,), -1, jnp.int32))
        pltpu.sync_copy(stv, outh.at[pl.ds(o, L)])
        return 0

    lax.fori_loop(0, C // NS, out_body, 0)


_nms_sc = functools.partial(
    pl.kernel,
    out_type=jax.ShapeDtypeStruct((N_PAD,), jnp.int32),
    mesh=plsc.VectorSubcoreMesh(core_axis_name="c", subcore_axis_name="s",
                                num_cores=1, num_subcores=NS),
    scratch_types=[
        pltpu.VMEM((N_PAD,), jnp.float32),
        pltpu.VMEM((N_PAD,), jnp.float32),
        pltpu.VMEM((N_PAD,), jnp.float32),
        pltpu.VMEM((N_PAD,), jnp.float32),
        pltpu.VMEM((N_PAD + L,), jnp.float32),
        pltpu.VMEM((N_PAD,), jnp.int32),
        pltpu.VMEM((L,), jnp.int32),
        pltpu.VMEM_SHARED((B,), jnp.float32),
    ],
)(_nms_body)


def kernel(boxes, scores, class_ids):
    max_c = boxes.max()
    offs = class_ids.astype(boxes.dtype) * (max_c + 1.0)
    b = boxes + offs[:, None]
    order = jnp.argsort(-scores)
    bs = b[order]
    padc = jnp.full((N_PAD - N,), PAD, jnp.float32)
    x1 = jnp.concatenate([bs[:, 0], padc])
    y1 = jnp.concatenate([bs[:, 1], padc])
    x2 = jnp.concatenate([bs[:, 2], padc])
    y2 = jnp.concatenate([bs[:, 3], padc])
    ordp = jnp.concatenate(
        [order.astype(jnp.int32), jnp.full((N_PAD - N,), -1, jnp.int32)])
    out = _nms_sc(x1, y1, x2, y2, ordp)
    return out[:N]


# speedup vs baseline: 1.0810x; 1.0810x over previous
import functools

import jax
import jax.numpy as jnp
from jax import lax
from jax.experimental import pallas as pl
from jax.experimental.pallas import tpu as pltpu
from jax.experimental.pallas import tpu_sc as plsc

N = 5000
TH = 0.5
L = 16
NS = 16
B = 64
N_PAD = 5120
NB = N_PAD // B
NCB = B // L
C = N_PAD // L
PAD = -1e30


def _nms_body(x1h, y1h, x2h, y2h, oh, outh,
              x1v, y1v, x2v, y2v, arv, alv, odv, stv, shv):
    sid = lax.axis_index("s")

    pltpu.sync_copy(x1h, x1v)
    pltpu.sync_copy(y1h, y1v)
    pltpu.sync_copy(x2h, x2v)
    pltpu.sync_copy(y2h, y2v)
    pltpu.sync_copy(oh, odv)

    def init_c(c, _):
        o = c * L
        w = jnp.maximum(x2v[pl.ds(o, L)] - x1v[pl.ds(o, L)], 0.0)
        h = jnp.maximum(y2v[pl.ds(o, L)] - y1v[pl.ds(o, L)], 0.0)
        arv[pl.ds(o, L)] = w * h
        alv[pl.ds(o, L)] = jnp.full((L,), 1.0, jnp.float32)
        return 0

    lax.fori_loop(0, N_PAD // L, init_c, 0)
    lanes = lax.iota(jnp.int32, L)

    def block_body(k, _):
        base = k * B
        j = jnp.remainder(sid - k * NCB, NS)

        @pl.when(j < NCB)
        def _():
            pltpu.sync_copy(alv.at[pl.ds(base + j * L, L)],
                            shv.at[pl.ds(j * L, L)])

        plsc.subcore_barrier()
        pltpu.sync_copy(shv, alv.at[pl.ds(base, B)])
        plsc.subcore_barrier()

        start = (k + 1) * NCB
        c0 = start + jnp.remainder(sid - start, NS)

        def chunk_body(cc, _):
            ci = k * NCB + cc
            oi = ci * L
            x1c = x1v[pl.ds(oi, L)]
            y1c = y1v[pl.ds(oi, L)]
            x2c = x2v[pl.ds(oi, L)]
            y2c = y2v[pl.ds(oi, L)]
            arc = arv[pl.ds(oi, L)]

            def lane_body(li, _):
                gi = oi + li
                a_i = alv[pl.ds(gi, L)][0]

                @pl.when(a_i > 0.0)
                def _():
                    liv = jnp.full((L,), li, jnp.int32)

                    def tk(vec):
                        return vec.at[liv].get(mode="promise_in_bounds")

                    x1i = tk(x1c)
                    y1i = tk(y1c)
                    x2i = tk(x2c)
                    y2i = tk(y2c)
                    ari = tk(arc)

                    def sup_chunk(c, extra=None):
                        o = c * L
                        ix1 = jnp.maximum(x1v[pl.ds(o, L)], x1i)
                        iy1 = jnp.maximum(y1v[pl.ds(o, L)], y1i)
                        ix2 = jnp.minimum(x2v[pl.ds(o, L)], x2i)
                        iy2 = jnp.minimum(y2v[pl.ds(o, L)], y2i)
                        inter = (jnp.maximum(ix2 - ix1, 0.0)
                                 * jnp.maximum(iy2 - iy1, 0.0))
                        union = arv[pl.ds(o, L)] + ari - inter
                        sup = inter > union * TH
                        if extra is not None:
                            sup = jnp.logical_and(sup, extra)
                        alv[pl.ds(o, L)] = jnp.where(sup, 0.0,
                                                     alv[pl.ds(o, L)])

                    sup_chunk(ci, lanes > li)

                    @plsc.parallel_loop(ci + 1, (k + 1) * NCB, unroll=4)
                    def _rest(c):
                        sup_chunk(c)

                    @plsc.parallel_loop(c0, C, step=NS, unroll=4)
                    def _tail(c):
                        sup_chunk(c)

                return 0

            lax.fori_loop(0, L, lane_body, 0)
            return 0

        lax.fori_loop(0, NCB, chunk_body, 0)
        return 0

    lax.fori_loop(0, NB, block_body, 0)

    def out_body(m, _):
        o = (m * NS + sid) * L
        stv[...] = jnp.where(alv[pl.ds(o, L)] > 0.0, odv[pl.ds(o, L)],
                             jnp.full((L,), -1, jnp.int32))
        pltpu.sync_copy(stv, outh.at[pl.ds(o, L)])
        return 0

    lax.fori_loop(0, C // NS, out_body, 0)


_nms_sc = functools.partial(
    pl.kernel,
    out_type=jax.ShapeDtypeStruct((N_PAD,), jnp.int32),
    mesh=plsc.VectorSubcoreMesh(core_axis_name="c", subcore_axis_name="s",
                                num_cores=1, num_subcores=NS),
    scratch_types=[
        pltpu.VMEM((N_PAD,), jnp.float32),
        pltpu.VMEM((N_PAD,), jnp.float32),
        pltpu.VMEM((N_PAD,), jnp.float32),
        pltpu.VMEM((N_PAD,), jnp.float32),
        pltpu.VMEM((N_PAD,), jnp.float32),
        pltpu.VMEM((N_PAD + L,), jnp.float32),
        pltpu.VMEM((N_PAD,), jnp.int32),
        pltpu.VMEM((L,), jnp.int32),
        pltpu.VMEM_SHARED((B,), jnp.float32),
    ],
)(_nms_body)


def kernel(boxes, scores, class_ids):
    max_c = boxes.max()
    offs = class_ids.astype(boxes.dtype) * (max_c + 1.0)
    b = boxes + offs[:, None]
    order = jnp.argsort(-scores)
    bs = b[order]
    padc = jnp.full((N_PAD - N,), PAD, jnp.float32)
    x1 = jnp.concatenate([bs[:, 0], padc])
    y1 = jnp.concatenate([bs[:, 1], padc])
    x2 = jnp.concatenate([bs[:, 2], padc])
    y2 = jnp.concatenate([bs[:, 3], padc])
    ordp = jnp.concatenate(
        [order.astype(jnp.int32), jnp.full((N_PAD - N,), -1, jnp.int32)])
    out = _nms_sc(x1, y1, x2, y2, ordp)
    return out[:N]


# speedup vs baseline: 1.1280x; 1.0435x over previous
import functools

import jax
import jax.numpy as jnp
from jax import lax
from jax.experimental import pallas as pl
from jax.experimental.pallas import tpu as pltpu
from jax.experimental.pallas import tpu_sc as plsc

N = 5000
TH = 0.5
L = 16
NS = 16
B = 32
N_PAD = 5120
NB = N_PAD // B
NCB = B // L
C = N_PAD // L
PAD = -1e30


def _nms_body(x1h, y1h, x2h, y2h, oh, outh,
              x1v, y1v, x2v, y2v, arv, alv, odv, stv, shv):
    sid = lax.axis_index("s")

    pltpu.sync_copy(x1h, x1v)
    pltpu.sync_copy(y1h, y1v)
    pltpu.sync_copy(x2h, x2v)
    pltpu.sync_copy(y2h, y2v)
    pltpu.sync_copy(oh, odv)

    def init_c(c, _):
        o = c * L
        w = jnp.maximum(x2v[pl.ds(o, L)] - x1v[pl.ds(o, L)], 0.0)
        h = jnp.maximum(y2v[pl.ds(o, L)] - y1v[pl.ds(o, L)], 0.0)
        arv[pl.ds(o, L)] = w * h
        alv[pl.ds(o, L)] = jnp.full((L,), 1.0, jnp.float32)
        return 0

    lax.fori_loop(0, N_PAD // L, init_c, 0)
    lanes = lax.iota(jnp.int32, L)

    def block_body(k, _):
        base = k * B
        j = jnp.remainder(sid - k * NCB, NS)

        @pl.when(j < NCB)
        def _():
            pltpu.sync_copy(alv.at[pl.ds(base + j * L, L)],
                            shv.at[pl.ds(j * L, L)])

        plsc.subcore_barrier()
        pltpu.sync_copy(shv, alv.at[pl.ds(base, B)])
        plsc.subcore_barrier()

        start = (k + 1) * NCB
        c0 = start + jnp.remainder(sid - start, NS)

        def chunk_body(cc, _):
            ci = k * NCB + cc
            oi = ci * L
            x1c = x1v[pl.ds(oi, L)]
            y1c = y1v[pl.ds(oi, L)]
            x2c = x2v[pl.ds(oi, L)]
            y2c = y2v[pl.ds(oi, L)]
            arc = arv[pl.ds(oi, L)]

            def lane_body(li, _):
                gi = oi + li
                a_i = alv[pl.ds(gi, L)][0]

                @pl.when(a_i > 0.0)
                def _():
                    liv = jnp.full((L,), li, jnp.int32)

                    def tk(vec):
                        return vec.at[liv].get(mode="promise_in_bounds")

                    x1i = tk(x1c)
                    y1i = tk(y1c)
                    x2i = tk(x2c)
                    y2i = tk(y2c)
                    ari = tk(arc)

                    def sup_chunk(c, extra=None):
                        o = c * L
                        ix1 = jnp.maximum(x1v[pl.ds(o, L)], x1i)
                        iy1 = jnp.maximum(y1v[pl.ds(o, L)], y1i)
                        ix2 = jnp.minimum(x2v[pl.ds(o, L)], x2i)
                        iy2 = jnp.minimum(y2v[pl.ds(o, L)], y2i)
                        inter = (jnp.maximum(ix2 - ix1, 0.0)
                                 * jnp.maximum(iy2 - iy1, 0.0))
                        union = arv[pl.ds(o, L)] + ari - inter
                        sup = inter > union * TH
                        if extra is not None:
                            sup = jnp.logical_and(sup, extra)
                        alv[pl.ds(o, L)] = jnp.where(sup, 0.0,
                                                     alv[pl.ds(o, L)])

                    sup_chunk(ci, lanes > li)

                    @plsc.parallel_loop(ci + 1, (k + 1) * NCB, unroll=4)
                    def _rest(c):
                        sup_chunk(c)

                    @plsc.parallel_loop(c0, C, step=NS, unroll=4)
                    def _tail(c):
                        sup_chunk(c)

                return 0

            lax.fori_loop(0, L, lane_body, 0)
            return 0

        lax.fori_loop(0, NCB, chunk_body, 0)
        return 0

    lax.fori_loop(0, NB, block_body, 0)

    def out_body(m, _):
        o = (m * NS + sid) * L
        stv[...] = jnp.where(alv[pl.ds(o, L)] > 0.0, odv[pl.ds(o, L)],
                             jnp.full((L,), -1, jnp.int32))
        pltpu.sync_copy(stv, outh.at[pl.ds(o, L)])
        return 0

    lax.fori_loop(0, C // NS, out_body, 0)


_nms_sc = functools.partial(
    pl.kernel,
    out_type=jax.ShapeDtypeStruct((N_PAD,), jnp.int32),
    mesh=plsc.VectorSubcoreMesh(core_axis_name="c", subcore_axis_name="s",
                                num_cores=1, num_subcores=NS),
    scratch_types=[
        pltpu.VMEM((N_PAD,), jnp.float32),
        pltpu.VMEM((N_PAD,), jnp.float32),
        pltpu.VMEM((N_PAD,), jnp.float32),
        pltpu.VMEM((N_PAD,), jnp.float32),
        pltpu.VMEM((N_PAD,), jnp.float32),
        pltpu.VMEM((N_PAD + L,), jnp.float32),
        pltpu.VMEM((N_PAD,), jnp.int32),
        pltpu.VMEM((L,), jnp.int32),
        pltpu.VMEM_SHARED((B,), jnp.float32),
    ],
)(_nms_body)


def kernel(boxes, scores, class_ids):
    max_c = boxes.max()
    offs = class_ids.astype(boxes.dtype) * (max_c + 1.0)
    b = boxes + offs[:, None]
    order = jnp.argsort(-scores)
    bs = b[order]
    padc = jnp.full((N_PAD - N,), PAD, jnp.float32)
    x1 = jnp.concatenate([bs[:, 0], padc])
    y1 = jnp.concatenate([bs[:, 1], padc])
    x2 = jnp.concatenate([bs[:, 2], padc])
    y2 = jnp.concatenate([bs[:, 3], padc])
    ordp = jnp.concatenate(
        [order.astype(jnp.int32), jnp.full((N_PAD - N,), -1, jnp.int32)])
    out = _nms_sc(x1, y1, x2, y2, ordp)
    return out[:N]


# speedup vs baseline: 1.1358x; 1.0069x over previous
import functools

import jax
import jax.numpy as jnp
from jax import lax
from jax.experimental import pallas as pl
from jax.experimental.pallas import tpu as pltpu
from jax.experimental.pallas import tpu_sc as plsc

N = 5000
TH = 0.5
L = 16
NS = 16
B = 16
N_PAD = 5120
NB = N_PAD // B
NCB = B // L
C = N_PAD // L
PAD = -1e30


def _nms_body(x1h, y1h, x2h, y2h, oh, outh,
              x1v, y1v, x2v, y2v, arv, alv, odv, stv, shv):
    sid = lax.axis_index("s")

    pltpu.sync_copy(x1h, x1v)
    pltpu.sync_copy(y1h, y1v)
    pltpu.sync_copy(x2h, x2v)
    pltpu.sync_copy(y2h, y2v)
    pltpu.sync_copy(oh, odv)

    def init_c(c, _):
        o = c * L
        w = jnp.maximum(x2v[pl.ds(o, L)] - x1v[pl.ds(o, L)], 0.0)
        h = jnp.maximum(y2v[pl.ds(o, L)] - y1v[pl.ds(o, L)], 0.0)
        arv[pl.ds(o, L)] = w * h
        alv[pl.ds(o, L)] = jnp.full((L,), 1.0, jnp.float32)
        return 0

    lax.fori_loop(0, N_PAD // L, init_c, 0)
    lanes = lax.iota(jnp.int32, L)

    def block_body(k, _):
        base = k * B
        j = jnp.remainder(sid - k * NCB, NS)

        @pl.when(j < NCB)
        def _():
            pltpu.sync_copy(alv.at[pl.ds(base + j * L, L)],
                            shv.at[pl.ds(j * L, L)])

        plsc.subcore_barrier()
        pltpu.sync_copy(shv, alv.at[pl.ds(base, B)])
        plsc.subcore_barrier()

        start = (k + 1) * NCB
        c0 = start + jnp.remainder(sid - start, NS)

        def chunk_body(cc, _):
            ci = k * NCB + cc
            oi = ci * L
            x1c = x1v[pl.ds(oi, L)]
            y1c = y1v[pl.ds(oi, L)]
            x2c = x2v[pl.ds(oi, L)]
            y2c = y2v[pl.ds(oi, L)]
            arc = arv[pl.ds(oi, L)]

            def lane_body(li, _):
                gi = oi + li
                a_i = alv[pl.ds(gi, L)][0]

                @pl.when(a_i > 0.0)
                def _():
                    liv = jnp.full((L,), li, jnp.int32)

                    def tk(vec):
                        return vec.at[liv].get(mode="promise_in_bounds")

                    x1i = tk(x1c)
                    y1i = tk(y1c)
                    x2i = tk(x2c)
                    y2i = tk(y2c)
                    ari = tk(arc)

                    def sup_chunk(c, extra=None):
                        o = c * L
                        ix1 = jnp.maximum(x1v[pl.ds(o, L)], x1i)
                        iy1 = jnp.maximum(y1v[pl.ds(o, L)], y1i)
                        ix2 = jnp.minimum(x2v[pl.ds(o, L)], x2i)
                        iy2 = jnp.minimum(y2v[pl.ds(o, L)], y2i)
                        inter = (jnp.maximum(ix2 - ix1, 0.0)
                                 * jnp.maximum(iy2 - iy1, 0.0))
                        union = arv[pl.ds(o, L)] + ari - inter
                        sup = inter > union * TH
                        if extra is not None:
                            sup = jnp.logical_and(sup, extra)
                        alv[pl.ds(o, L)] = jnp.where(sup, 0.0,
                                                     alv[pl.ds(o, L)])

                    sup_chunk(ci, lanes > li)

                    @plsc.parallel_loop(ci + 1, (k + 1) * NCB, unroll=4)
                    def _rest(c):
                        sup_chunk(c)

                    @plsc.parallel_loop(c0, C, step=NS, unroll=4)
                    def _tail(c):
                        sup_chunk(c)

                return 0

            lax.fori_loop(0, L, lane_body, 0)
            return 0

        lax.fori_loop(0, NCB, chunk_body, 0)
        return 0

    lax.fori_loop(0, NB, block_body, 0)

    def out_body(m, _):
        o = (m * NS + sid) * L
        stv[...] = jnp.where(alv[pl.ds(o, L)] > 0.0, odv[pl.ds(o, L)],
                             jnp.full((L,), -1, jnp.int32))
        pltpu.sync_copy(stv, outh.at[pl.ds(o, L)])
        return 0

    lax.fori_loop(0, C // NS, out_body, 0)


_nms_sc = functools.partial(
    pl.kernel,
    out_type=jax.ShapeDtypeStruct((N_PAD,), jnp.int32),
    mesh=plsc.VectorSubcoreMesh(core_axis_name="c", subcore_axis_name="s",
                                num_cores=1, num_subcores=NS),
    scratch_types=[
        pltpu.VMEM((N_PAD,), jnp.float32),
        pltpu.VMEM((N_PAD,), jnp.float32),
        pltpu.VMEM((N_PAD,), jnp.float32),
        pltpu.VMEM((N_PAD,), jnp.float32),
        pltpu.VMEM((N_PAD,), jnp.float32),
        pltpu.VMEM((N_PAD + L,), jnp.float32),
        pltpu.VMEM((N_PAD,), jnp.int32),
        pltpu.VMEM((L,), jnp.int32),
        pltpu.VMEM_SHARED((B,), jnp.float32),
    ],
)(_nms_body)


def kernel(boxes, scores, class_ids):
    max_c = boxes.max()
    offs = class_ids.astype(boxes.dtype) * (max_c + 1.0)
    b = boxes + offs[:, None]
    order = jnp.argsort(-scores)
    bs = b[order]
    padc = jnp.full((N_PAD - N,), PAD, jnp.float32)
    x1 = jnp.concatenate([bs[:, 0], padc])
    y1 = jnp.concatenate([bs[:, 1], padc])
    x2 = jnp.concatenate([bs[:, 2], padc])
    y2 = jnp.concatenate([bs[:, 3], padc])
    ordp = jnp.concatenate(
        [order.astype(jnp.int32), jnp.full((N_PAD - N,), -1, jnp.int32)])
    out = _nms_sc(x1, y1, x2, y2, ordp)
    return out[:N]


# speedup vs baseline: 1.1795x; 1.0385x over previous
import functools

import jax
import jax.numpy as jnp
from jax import lax
from jax.experimental import pallas as pl
from jax.experimental.pallas import tpu as pltpu
from jax.experimental.pallas import tpu_sc as plsc

N = 5000
TH = 0.5
L = 16
NS = 16
N_PAD = 5120
C = N_PAD // L
NBLK = (N + L - 1) // L
PAD = -1e30


def _nms_body(x1h, y1h, x2h, y2h, oh, outh,
              x1v, y1v, x2v, y2v, arv, alv, odv, stv, shv):
    sid = lax.axis_index("s")

    pltpu.sync_copy(x1h, x1v)
    pltpu.sync_copy(y1h, y1v)
    pltpu.sync_copy(x2h, x2v)
    pltpu.sync_copy(y2h, y2v)
    pltpu.sync_copy(oh, odv)

    def init_c(c, _):
        o = c * L
        w = jnp.maximum(x2v[pl.ds(o, L)] - x1v[pl.ds(o, L)], 0.0)
        h = jnp.maximum(y2v[pl.ds(o, L)] - y1v[pl.ds(o, L)], 0.0)
        arv[pl.ds(o, L)] = w * h
        alv[pl.ds(o, L)] = jnp.full((L,), 1.0, jnp.float32)
        return 0

    lax.fori_loop(0, N_PAD // L, init_c, 0)
    lanes = lax.iota(jnp.int32, L)

    def block_body(ci, _):
        oi = ci * L
        own = jnp.remainder(ci, NS) == sid
        buf = jnp.remainder(ci, 2) * L

        @pl.when(own)
        def _():
            pltpu.sync_copy(alv.at[pl.ds(oi, L)], shv.at[pl.ds(buf, L)])

        plsc.subcore_barrier()

        @pl.when(jnp.logical_not(own))
        def _():
            pltpu.sync_copy(shv.at[pl.ds(buf, L)], alv.at[pl.ds(oi, L)])

        x1c = x1v[pl.ds(oi, L)]
        y1c = y1v[pl.ds(oi, L)]
        x2c = x2v[pl.ds(oi, L)]
        y2c = y2v[pl.ds(oi, L)]
        arc = arv[pl.ds(oi, L)]
        c0 = ci + 1 + jnp.remainder(sid - ci - 1, NS)

        def lane_body(li, _):
            gi = oi + li
            a_i = alv[pl.ds(gi, L)][0]

            @pl.when(a_i > 0.0)
            def _():
                liv = jnp.full((L,), li, jnp.int32)

                def tk(vec):
                    return vec.at[liv].get(mode="promise_in_bounds")

                x1i = tk(x1c)
                y1i = tk(y1c)
                x2i = tk(x2c)
                y2i = tk(y2c)
                ari = tk(arc)

                def sup_chunk(c, extra=None):
                    o = c * L
                    ix1 = jnp.maximum(x1v[pl.ds(o, L)], x1i)
                    iy1 = jnp.maximum(y1v[pl.ds(o, L)], y1i)
                    ix2 = jnp.minimum(x2v[pl.ds(o, L)], x2i)
                    iy2 = jnp.minimum(y2v[pl.ds(o, L)], y2i)
                    inter = (jnp.maximum(ix2 - ix1, 0.0)
                             * jnp.maximum(iy2 - iy1, 0.0))
                    union = arv[pl.ds(o, L)] + ari - inter
                    sup = inter > union * TH
                    if extra is not None:
                        sup = jnp.logical_and(sup, extra)
                    alv[pl.ds(o, L)] = jnp.where(sup, 0.0,
                                                 alv[pl.ds(o, L)])

                sup_chunk(ci, lanes > li)

                @plsc.parallel_loop(c0, C, step=NS, unroll=4)
                def _tail(c):
                    sup_chunk(c)

            return 0

        lax.fori_loop(0, L, lane_body, 0)
        return 0

    lax.fori_loop(0, NBLK, block_body, 0)

    def out_body(m, _):
        o = (m * NS + sid) * L
        stv[...] = jnp.where(alv[pl.ds(o, L)] > 0.0, odv[pl.ds(o, L)],
                             jnp.full((L,), -1, jnp.int32))
        pltpu.sync_copy(stv, outh.at[pl.ds(o, L)])
        return 0

    lax.fori_loop(0, C // NS, out_body, 0)


_nms_sc = functools.partial(
    pl.kernel,
    out_type=jax.ShapeDtypeStruct((N_PAD,), jnp.int32),
    mesh=plsc.VectorSubcoreMesh(core_axis_name="c", subcore_axis_name="s",
                                num_cores=1, num_subcores=NS),
    scratch_types=[
        pltpu.VMEM((N_PAD,), jnp.float32),
        pltpu.VMEM((N_PAD,), jnp.float32),
        pltpu.VMEM((N_PAD,), jnp.float32),
        pltpu.VMEM((N_PAD,), jnp.float32),
        pltpu.VMEM((N_PAD,), jnp.float32),
        pltpu.VMEM((N_PAD + L,), jnp.float32),
        pltpu.VMEM((N_PAD,), jnp.int32),
        pltpu.VMEM((L,), jnp.int32),
        pltpu.VMEM_SHARED((2 * L,), jnp.float32),
    ],
)(_nms_body)


def kernel(boxes, scores, class_ids):
    max_c = boxes.max()
    offs = class_ids.astype(boxes.dtype) * (max_c + 1.0)
    b = boxes + offs[:, None]
    order = jnp.argsort(-scores)
    bs = b[order]
    padc = jnp.full((N_PAD - N,), PAD, jnp.float32)
    x1 = jnp.concatenate([bs[:, 0], padc])
    y1 = jnp.concatenate([bs[:, 1], padc])
    x2 = jnp.concatenate([bs[:, 2], padc])
    y2 = jnp.concatenate([bs[:, 3], padc])
    ordp = jnp.concatenate(
        [order.astype(jnp.int32), jnp.full((N_PAD - N,), -1, jnp.int32)])
    out = _nms_sc(x1, y1, x2, y2, ordp)
    return out[:N]


# speedup vs baseline: 1.8545x; 1.5722x over previous
import functools

import jax
import jax.numpy as jnp
from jax import lax
from jax.experimental import pallas as pl
from jax.experimental.pallas import tpu as pltpu
from jax.experimental.pallas import tpu_sc as plsc

N = 5000
NUM_CLASSES = 80
TH = 0.5
L = 16
NS = 16
GPT = NUM_CLASSES // NS
C2 = 400
CAP = C2 * L
NCP = NUM_CLASSES + L
PAD = -1e30


def _nms_body(x1h, y1h, x2h, y2h, sgh, nch, outh,
              x1v, y1v, x2v, y2v, arv, alv, sgv, ncv, stv):
    sid = lax.axis_index("s")

    pltpu.sync_copy(x1h, x1v)
    pltpu.sync_copy(y1h, y1v)
    pltpu.sync_copy(x2h, x2v)
    pltpu.sync_copy(y2h, y2v)
    pltpu.sync_copy(sgh, sgv)
    pltpu.sync_copy(nch, ncv)

    lanes = lax.iota(jnp.int32, L)

    def class_body(t, _):
        g = t * NS + sid
        sc = sgv[pl.ds(g, L)][0]
        nc = ncv[pl.ds(g, L)][0]
        end = sc + nc

        def init_c(c, _):
            o = c * L
            w = jnp.maximum(x2v[pl.ds(o, L)] - x1v[pl.ds(o, L)], 0.0)
            h = jnp.maximum(y2v[pl.ds(o, L)] - y1v[pl.ds(o, L)], 0.0)
            a = w * h
            arv[pl.ds(o, L)] = a
            alv[pl.ds(o, L)] = jnp.where(a > 0.0, 1.0, 0.0)
            return 0

        lax.fori_loop(sc, end, init_c, 0)

        def chunk_body(ci, _):
            oi = ci * L
            x1c = x1v[pl.ds(oi, L)]
            y1c = y1v[pl.ds(oi, L)]
            x2c = x2v[pl.ds(oi, L)]
            y2c = y2v[pl.ds(oi, L)]
            arc = arv[pl.ds(oi, L)]

            def lane_body(li, _):
                gi = oi + li
                a_i = alv[pl.ds(gi, L)][0]

                @pl.when(a_i > 0.0)
                def _():
                    liv = jnp.full((L,), li, jnp.int32)

                    def tk(vec):
                        return vec.at[liv].get(mode="promise_in_bounds")

                    x1i = tk(x1c)
                    y1i = tk(y1c)
                    x2i = tk(x2c)
                    y2i = tk(y2c)
                    ari = tk(arc)

                    def sup_chunk(c, extra=None):
                        o = c * L
                        ix1 = jnp.maximum(x1v[pl.ds(o, L)], x1i)
                        iy1 = jnp.maximum(y1v[pl.ds(o, L)], y1i)
                        ix2 = jnp.minimum(x2v[pl.ds(o, L)], x2i)
                        iy2 = jnp.minimum(y2v[pl.ds(o, L)], y2i)
                        inter = (jnp.maximum(ix2 - ix1, 0.0)
                                 * jnp.maximum(iy2 - iy1, 0.0))
                        union = arv[pl.ds(o, L)] + ari - inter
                        sup = inter > union * TH
                        if extra is not None:
                            sup = jnp.logical_and(sup, extra)
                        alv[pl.ds(o, L)] = jnp.where(sup, 0.0,
                                                     alv[pl.ds(o, L)])

                    sup_chunk(ci, lanes > li)

                    @plsc.parallel_loop(ci + 1, end, unroll=2)
                    def _tail(c):
                        sup_chunk(c)

                return 0

            lax.fori_loop(0, L, lane_body, 0)

            stv[...] = jnp.where(alv[pl.ds(oi, L)] > 0.0,
                                 jnp.full((L,), 1, jnp.int32),
                                 jnp.full((L,), 0, jnp.int32))
            pltpu.sync_copy(stv, outh.at[pl.ds(oi, L)])
            return 0

        lax.fori_loop(sc, end, chunk_body, 0)
        return 0

    lax.fori_loop(0, GPT, class_body, 0)


_nms_sc = functools.partial(
    pl.kernel,
    out_type=jax.ShapeDtypeStruct((CAP,), jnp.int32),
    mesh=plsc.VectorSubcoreMesh(core_axis_name="c", subcore_axis_name="s",
                                num_cores=1, num_subcores=NS),
    scratch_types=[
        pltpu.VMEM((CAP,), jnp.float32),
        pltpu.VMEM((CAP,), jnp.float32),
        pltpu.VMEM((CAP,), jnp.float32),
        pltpu.VMEM((CAP,), jnp.float32),
        pltpu.VMEM((CAP,), jnp.float32),
        pltpu.VMEM((CAP + L,), jnp.float32),
        pltpu.VMEM((NCP,), jnp.int32),
        pltpu.VMEM((NCP,), jnp.int32),
        pltpu.VMEM((L,), jnp.int32),
    ],
)(_nms_body)


def kernel(boxes, scores, class_ids):
    max_c = boxes.max()
    cls = class_ids.astype(jnp.int32)
    offs = class_ids.astype(boxes.dtype) * (max_c + 1.0)
    b = boxes + offs[:, None]
    order = jnp.argsort(-scores)
    bs = b[order]
    cls_r = cls[order]
    perm = jnp.argsort(cls_r, stable=True)
    cls_s = cls_r[perm]
    grouped = bs[perm]

    cnt = jnp.bincount(cls, length=NUM_CLASSES)
    nch = (cnt + L - 1) // L
    seg_c = jnp.concatenate([jnp.zeros((1,), jnp.int32),
                             jnp.cumsum(nch)[:-1].astype(jnp.int32)])
    seg_start = seg_c * L
    unp_start = jnp.concatenate([jnp.zeros((1,), jnp.int32),
                                 jnp.cumsum(cnt)[:-1].astype(jnp.int32)])
    pp = seg_start[cls_s] + (jnp.arange(N, dtype=jnp.int32)
                             - unp_start[cls_s])

    padf = jnp.full((CAP,), PAD, jnp.float32)
    x1 = padf.at[pp].set(grouped[:, 0])
    y1 = padf.at[pp].set(grouped[:, 1])
    x2 = padf.at[pp].set(grouped[:, 2])
    y2 = padf.at[pp].set(grouped[:, 3])
    padi = jnp.zeros((NCP - NUM_CLASSES,), jnp.int32)
    sg = jnp.concatenate([seg_c.astype(jnp.int32), padi])
    nc = jnp.concatenate([nch.astype(jnp.int32), padi])

    keep01 = _nms_sc(x1, y1, x2, y2, sg, nc)
    keep_rank = jnp.zeros((N,), jnp.int32).at[perm].set(keep01[pp])
    return jnp.where(keep_rank > 0, order, -1)


# speedup vs baseline: 2.6673x; 1.4383x over previous
import functools

import jax
import jax.numpy as jnp
from jax import lax
from jax.experimental import pallas as pl
from jax.experimental.pallas import tpu as pltpu
from jax.experimental.pallas import tpu_sc as plsc

N = 5000
NUM_CLASSES = 80
TH = 0.5
L = 16
NS = 16
GPT = NUM_CLASSES // NS
NPIN = N + L
C2 = 400
CAP = C2 * L
NCP = NUM_CLASSES + L
PAD = -1e30


def _nms_body(x1h, y1h, x2h, y2h, sth, lnh, sgh, outh,
              x1v, y1v, x2v, y2v, arv, alv, stv_, lnv, sgv, ov):
    sid = lax.axis_index("s")

    pltpu.sync_copy(x1h, x1v)
    pltpu.sync_copy(y1h, y1v)
    pltpu.sync_copy(x2h, x2v)
    pltpu.sync_copy(y2h, y2v)
    pltpu.sync_copy(sth, stv_)
    pltpu.sync_copy(lnh, lnv)
    pltpu.sync_copy(sgh, sgv)

    lanes = lax.iota(jnp.int32, L)

    def class_body(t, _):
        g = t * NS + sid
        s = stv_[pl.ds(g, L)][0]
        ln = lnv[pl.ds(g, L)][0]
        oc = sgv[pl.ds(g, L)][0]
        nc = (ln + L - 1) // L
        end = s + nc * L

        def init_c(u, _):
            o = s + u * L
            w = jnp.maximum(x2v[pl.ds(o, L)] - x1v[pl.ds(o, L)], 0.0)
            h = jnp.maximum(y2v[pl.ds(o, L)] - y1v[pl.ds(o, L)], 0.0)
            a = w * h
            arv[pl.ds(o, L)] = a
            alv[pl.ds(o, L)] = jnp.where(a > 0.0, 1.0, 0.0)
            return 0

        lax.fori_loop(0, nc, init_c, 0)

        def chunk_body(u, _):
            oi = s + u * L
            x1c = x1v[pl.ds(oi, L)]
            y1c = y1v[pl.ds(oi, L)]
            x2c = x2v[pl.ds(oi, L)]
            y2c = y2v[pl.ds(oi, L)]
            arc = arv[pl.ds(oi, L)]

            def lane_body(li, _):
                gi = oi + li
                a_i = alv[pl.ds(gi, L)][0]

                @pl.when(a_i > 0.0)
                def _():
                    liv = jnp.full((L,), li, jnp.int32)

                    def tk(vec):
                        return vec.at[liv].get(mode="promise_in_bounds")

                    x1i = tk(x1c)
                    y1i = tk(y1c)
                    x2i = tk(x2c)
                    y2i = tk(y2c)
                    ari = tk(arc)

                    def sup_off(o, extra=None):
                        ix1 = jnp.maximum(x1v[pl.ds(o, L)], x1i)
                        iy1 = jnp.maximum(y1v[pl.ds(o, L)], y1i)
                        ix2 = jnp.minimum(x2v[pl.ds(o, L)], x2i)
                        iy2 = jnp.minimum(y2v[pl.ds(o, L)], y2i)
                        inter = (jnp.maximum(ix2 - ix1, 0.0)
                                 * jnp.maximum(iy2 - iy1, 0.0))
                        union = arv[pl.ds(o, L)] + ari - inter
                        sup = inter > union * TH
                        if extra is not None:
                            sup = jnp.logical_and(sup, extra)
                        alv[pl.ds(o, L)] = jnp.where(sup, 0.0,
                                                     alv[pl.ds(o, L)])

                    sup_off(oi, lanes > li)

                    @plsc.parallel_loop(oi + L, end, step=L, unroll=2)
                    def _tail(o):
                        sup_off(o)

                return 0

            lax.fori_loop(0, L, lane_body, 0)

            ov[...] = jnp.where(alv[pl.ds(oi, L)] > 0.0,
                                jnp.full((L,), 1, jnp.int32),
                                jnp.full((L,), 0, jnp.int32))
            pltpu.sync_copy(ov, outh.at[pl.ds((oc + u) * L, L)])
            return 0

        lax.fori_loop(0, nc, chunk_body, 0)
        return 0

    lax.fori_loop(0, GPT, class_body, 0)


_nms_sc = functools.partial(
    pl.kernel,
    out_type=jax.ShapeDtypeStruct((CAP,), jnp.int32),
    mesh=plsc.VectorSubcoreMesh(core_axis_name="c", subcore_axis_name="s",
                                num_cores=1, num_subcores=NS),
    scratch_types=[
        pltpu.VMEM((NPIN,), jnp.float32),
        pltpu.VMEM((NPIN,), jnp.float32),
        pltpu.VMEM((NPIN,), jnp.float32),
        pltpu.VMEM((NPIN,), jnp.float32),
        pltpu.VMEM((NPIN + L,), jnp.float32),
        pltpu.VMEM((NPIN + L,), jnp.float32),
        pltpu.VMEM((NCP,), jnp.int32),
        pltpu.VMEM((NCP,), jnp.int32),
        pltpu.VMEM((NCP,), jnp.int32),
        pltpu.VMEM((L,), jnp.int32),
    ],
)(_nms_body)


def kernel(boxes, scores, class_ids):
    max_c = boxes.max()
    cls = class_ids.astype(jnp.int32)
    offs = class_ids.astype(boxes.dtype) * (max_c + 1.0)
    b = boxes + offs[:, None]
    order = jnp.argsort(-scores)
    cls_r = cls[order]
    perm = jnp.argsort(cls_r, stable=True)
    ord2 = order[perm]
    grouped = b[ord2]

    cnt = jnp.bincount(cls, length=NUM_CLASSES).astype(jnp.int32)
    nch = (cnt + L - 1) // L
    seg_c = jnp.concatenate([jnp.zeros((1,), jnp.int32),
                             jnp.cumsum(nch)[:-1].astype(jnp.int32)])
    unp_start = jnp.concatenate([jnp.zeros((1,), jnp.int32),
                                 jnp.cumsum(cnt)[:-1].astype(jnp.int32)])

    padc = jnp.full((L,), PAD, jnp.float32)
    x1 = jnp.concatenate([grouped[:, 0], padc])
    y1 = jnp.concatenate([grouped[:, 1], padc])
    x2 = jnp.concatenate([grouped[:, 2], padc])
    y2 = jnp.concatenate([grouped[:, 3], padc])
    padi = jnp.zeros((NCP - NUM_CLASSES,), jnp.int32)
    stp = jnp.concatenate([unp_start, padi])
    lnp = jnp.concatenate([cnt, padi])
    sgp = jnp.concatenate([seg_c, padi])

    keep01 = _nms_sc(x1, y1, x2, y2, stp, lnp, sgp)

    cls_s = cls_r[perm]
    pp = seg_c[cls_s] * L + (jnp.arange(N, dtype=jnp.int32)
                             - unp_start[cls_s])
    kg = keep01[pp]
    vals = jnp.where(kg > 0, ord2, -1)
    return jnp.full((N,), -1, jnp.int32).at[perm].set(vals)


# speedup vs baseline: 3.1278x; 1.1726x over previous
import functools

import jax
import jax.numpy as jnp
from jax import lax
from jax.experimental import pallas as pl
from jax.experimental.pallas import tpu as pltpu
from jax.experimental.pallas import tpu_sc as plsc

N = 5000
NUM_CLASSES = 80
TH = 0.5
L = 16
NS = 16
GPT = NUM_CLASSES // NS
NPIN = N + L
C2 = 400
CAP = C2 * L
NCP = NUM_CLASSES + L
PAD = -1e30


def _nms_body(x1h, y1h, x2h, y2h, sth, lnh, sgh, outh,
              x1v, y1v, x2v, y2v, arv, alv, stv_, lnv, sgv, ov):
    sid = lax.axis_index("s")

    pltpu.sync_copy(x1h, x1v)
    pltpu.sync_copy(y1h, y1v)
    pltpu.sync_copy(x2h, x2v)
    pltpu.sync_copy(y2h, y2v)
    pltpu.sync_copy(sth, stv_)
    pltpu.sync_copy(lnh, lnv)
    pltpu.sync_copy(sgh, sgv)

    lanes = lax.iota(jnp.int32, L)

    def class_body(t, _):
        g = t * NS + sid
        s = stv_[pl.ds(g, L)][0]
        ln = lnv[pl.ds(g, L)][0]
        oc = sgv[pl.ds(g, L)][0]
        nc = (ln + L - 1) // L
        end = s + nc * L

        def init_c(u, _):
            o = s + u * L
            w = jnp.maximum(x2v[pl.ds(o, L)] - x1v[pl.ds(o, L)], 0.0)
            h = jnp.maximum(y2v[pl.ds(o, L)] - y1v[pl.ds(o, L)], 0.0)
            a = w * h
            arv[pl.ds(o, L)] = a
            alv[pl.ds(o, L)] = jnp.where(a > 0.0, 1.0, 0.0)
            return 0

        lax.fori_loop(0, nc, init_c, 0)

        def chunk_body(u, _):
            oi = s + u * L
            x1c = x1v[pl.ds(oi, L)]
            y1c = y1v[pl.ds(oi, L)]
            x2c = x2v[pl.ds(oi, L)]
            y2c = y2v[pl.ds(oi, L)]
            arc = arv[pl.ds(oi, L)]

            def lane_body(li, _):
                gi = oi + li
                a_i = alv[pl.ds(gi, L)][0]

                @pl.when(a_i > 0.0)
                def _():
                    liv = jnp.full((L,), li, jnp.int32)

                    def tk(vec):
                        return vec.at[liv].get(mode="promise_in_bounds")

                    x1i = tk(x1c)
                    y1i = tk(y1c)
                    x2i = tk(x2c)
                    y2i = tk(y2c)
                    ari = tk(arc)

                    def sup_off(o, extra=None):
                        ix1 = jnp.maximum(x1v[pl.ds(o, L)], x1i)
                        iy1 = jnp.maximum(y1v[pl.ds(o, L)], y1i)
                        ix2 = jnp.minimum(x2v[pl.ds(o, L)], x2i)
                        iy2 = jnp.minimum(y2v[pl.ds(o, L)], y2i)
                        inter = (jnp.maximum(ix2 - ix1, 0.0)
                                 * jnp.maximum(iy2 - iy1, 0.0))
                        union = arv[pl.ds(o, L)] + ari - inter
                        sup = inter > union * TH
                        if extra is not None:
                            sup = jnp.logical_and(sup, extra)
                        alv[pl.ds(o, L)] = jnp.where(sup, 0.0,
                                                     alv[pl.ds(o, L)])

                    sup_off(oi, lanes > li)

                    @plsc.parallel_loop(oi + L, end, step=L, unroll=2)
                    def _tail(o):
                        sup_off(o)

                return 0

            lax.fori_loop(0, L, lane_body, 0)

            ov[...] = jnp.where(alv[pl.ds(oi, L)] > 0.0,
                                jnp.full((L,), 1, jnp.int32),
                                jnp.full((L,), 0, jnp.int32))
            pltpu.sync_copy(ov, outh.at[pl.ds((oc + u) * L, L)])
            return 0

        lax.fori_loop(0, nc, chunk_body, 0)
        return 0

    lax.fori_loop(0, GPT, class_body, 0)


_nms_sc = functools.partial(
    pl.kernel,
    out_type=jax.ShapeDtypeStruct((CAP,), jnp.int32),
    mesh=plsc.VectorSubcoreMesh(core_axis_name="c", subcore_axis_name="s",
                                num_cores=1, num_subcores=NS),
    scratch_types=[
        pltpu.VMEM((NPIN,), jnp.float32),
        pltpu.VMEM((NPIN,), jnp.float32),
        pltpu.VMEM((NPIN,), jnp.float32),
        pltpu.VMEM((NPIN,), jnp.float32),
        pltpu.VMEM((NPIN + L,), jnp.float32),
        pltpu.VMEM((NPIN + L,), jnp.float32),
        pltpu.VMEM((NCP,), jnp.int32),
        pltpu.VMEM((NCP,), jnp.int32),
        pltpu.VMEM((NCP,), jnp.int32),
        pltpu.VMEM((L,), jnp.int32),
    ],
)(_nms_body)


def kernel(boxes, scores, class_ids):
    max_c = boxes.max()
    cls = class_ids.astype(jnp.int32)
    offs = class_ids.astype(boxes.dtype) * (max_c + 1.0)
    b = boxes + offs[:, None]
    order = jnp.argsort(-scores)
    cls_r = cls[order]
    perm = jnp.argsort(cls_r, stable=True)
    ord2 = order[perm]
    grouped = b[ord2]

    cnt = jnp.bincount(cls, length=NUM_CLASSES).astype(jnp.int32)
    nch = (cnt + L - 1) // L
    seg_c = jnp.concatenate([jnp.zeros((1,), jnp.int32),
                             jnp.cumsum(nch)[:-1].astype(jnp.int32)])
    unp_start = jnp.concatenate([jnp.zeros((1,), jnp.int32),
                                 jnp.cumsum(cnt)[:-1].astype(jnp.int32)])

    padc = jnp.full((L,), PAD, jnp.float32)
    x1 = jnp.concatenate([grouped[:, 0], padc])
    y1 = jnp.concatenate([grouped[:, 1], padc])
    x2 = jnp.concatenate([grouped[:, 2], padc])
    y2 = jnp.concatenate([grouped[:, 3], padc])
    padi = jnp.zeros((NCP - NUM_CLASSES,), jnp.int32)
    stp = jnp.concatenate([unp_start, padi])
    lnp = jnp.concatenate([cnt, padi])
    sgp = jnp.concatenate([seg_c, padi])

    keep01 = _nms_sc(x1, y1, x2, y2, stp, lnp, sgp)

    cls_s = cls_r[perm]
    delta = seg_c * L - unp_start
    pp = delta[cls_s] + jnp.arange(N, dtype=jnp.int32)
    kg = keep01[pp]
    vals = jnp.where(kg > 0, ord2, -1)
    return jnp.full((N,), -1, jnp.int32).at[perm].set(vals)
